# baseline (speedup 1.0000x reference)
import functools

import jax
import jax.numpy as jnp
from jax import lax
from jax.experimental import pallas as pl
from jax.experimental.pallas import tpu as pltpu
from jax.experimental.pallas import tpu_sc as plsc

NC = 2
NS = 16
LANES = 16
NW = NC * NS
CHUNK = 128
BN = 512


@functools.lru_cache(maxsize=None)
def _edge_pass(F, nd_pad, e_pad, gather):
    n_chunks = e_pad // (NW * CHUNK)
    nblocks = nd_pad // CHUNK
    vpr = F // LANES
    mesh = plsc.VectorSubcoreMesh(core_axis_name="c", subcore_axis_name="s",
                                  num_cores=NC, num_subcores=NS)

    scratch = [
        pltpu.VMEM((CHUNK,), jnp.int32),
        pltpu.VMEM((CHUNK, F), jnp.float32),
        pltpu.VMEM_SHARED((nd_pad, F), jnp.float32),
        pltpu.SemaphoreType.DMA,
    ]
    if gather:
        scratch.insert(0, pltpu.VMEM((CHUNK,), jnp.int32))

    def body(*refs):
        if gather:
            table, sidx, didx, out, sidx_v, didx_v, rows_v, acc, sem = refs
        else:
            didx, out, didx_v, rows_v, acc, sem = refs
        c = lax.axis_index("c")
        s = lax.axis_index("s")
        w = c * NS + s
        nmine = (nblocks - s + NS - 1) // NS

        def fill(val):
            def fb(i, _):
                rows_v[i // vpr, pl.ds((i % vpr) * LANES, LANES)] = jnp.full(
                    (LANES,), val, jnp.float32)
                return 0
            lax.fori_loop(0, CHUNK * vpr, fb, 0)

        fill(0.0)

        def zacc(i, _):
            pltpu.sync_copy(rows_v, acc.at[pl.ds((s + i * NS) * CHUNK, CHUNK)])
            return 0
        lax.fori_loop(0, nmine, zacc, 0)
        if not gather:
            fill(1.0)
        plsc.subcore_barrier()

        def chunk(t, _):
            base = (w * n_chunks + t) * CHUNK
            pltpu.sync_copy(didx.at[pl.ds(base, CHUNK)], didx_v)
            if gather:
                pltpu.sync_copy(sidx.at[pl.ds(base, CHUNK)], sidx_v)
                pltpu.async_copy(table.at[sidx_v], rows_v, sem).wait()
            pltpu.sync_copy(rows_v, acc.at[didx_v], add=True)
            return 0
        lax.fori_loop(0, n_chunks, chunk, 0)
        plsc.subcore_barrier()

        def wout(i, _):
            r0 = (s + i * NS) * CHUNK
            pltpu.sync_copy(acc.at[pl.ds(r0, CHUNK)], rows_v)
            pltpu.sync_copy(rows_v, out.at[c, pl.ds(r0, CHUNK)])
            return 0
        lax.fori_loop(0, nmine, wout, 0)

    return pl.kernel(
        body,
        out_type=jax.ShapeDtypeStruct((NC, nd_pad, F), jnp.float32),
        mesh=mesh,
        scratch_types=scratch,
        compiler_params=pltpu.CompilerParams(use_tc_tiling_on_sc=False),
    )


def _mm_table(h_parts, si, W, so):
    N = so.shape[0]
    Fin, Fout = W.shape
    two = len(h_parts) == 2
    use_si = si is not None

    def body(*refs):
        i = 0
        h = refs[0][...]
        i = 1
        if two:
            h = h + refs[1][...]
            i = 2
        if use_si:
            h = h * refs[i][...]
            i += 1
        Wv = refs[i][...]
        sov = refs[i + 1][...]
        refs[i + 2][...] = jnp.dot(
            h.astype(jnp.bfloat16), Wv.astype(jnp.bfloat16),
            preferred_element_type=jnp.float32) * sov

    in_specs = [pl.BlockSpec((BN, Fin), lambda i: (i, 0))
                for _ in range(len(h_parts))]
    args = list(h_parts)
    if use_si:
        in_specs.append(pl.BlockSpec((BN, 1), lambda i: (i, 0)))
        args.append(si)
    in_specs.append(pl.BlockSpec((Fin, Fout), lambda i: (0, 0)))
    in_specs.append(pl.BlockSpec((BN, 1), lambda i: (i, 0)))
    args += [W, so]
    return pl.pallas_call(
        body,
        grid=(pl.cdiv(N, BN),),
        in_specs=in_specs,
        out_specs=pl.BlockSpec((BN, Fout), lambda i: (i, 0)),
        out_shape=jax.ShapeDtypeStruct((N, Fout), jnp.float32),
    )(*args)


def _combine(q0, q1, xwp, dis, b, relu):
    N, F = xwp.shape

    def body(q0r, q1r, xr, dr, br, outr):
        v = (q0r[...] + q1r[...] + xr[...]) * dr[...] + br[...]
        if relu:
            v = jnp.maximum(v, 0.0)
        outr[...] = v

    return pl.pallas_call(
        body,
        grid=(pl.cdiv(N, BN),),
        in_specs=[
            pl.BlockSpec((BN, F), lambda i: (i, 0)),
            pl.BlockSpec((BN, F), lambda i: (i, 0)),
            pl.BlockSpec((BN, F), lambda i: (i, 0)),
            pl.BlockSpec((BN, 1), lambda i: (i, 0)),
            pl.BlockSpec((1, F), lambda i: (0, 0)),
        ],
        out_specs=pl.BlockSpec((BN, F), lambda i: (i, 0)),
        out_shape=jax.ShapeDtypeStruct((N, F), jnp.float32),
    )(q0, q1, xwp, dis, b.reshape(1, F))


def _gcn_ctl(x, src, dst, emask, W, b):
    N = x.shape[0]
    w = emask.astype(x.dtype)
    s = jnp.where(emask, src, 0)
    d = jnp.where(emask, dst, 0)
    deg = jnp.ones((N,), x.dtype).at[d].add(w)
    dis = lax.rsqrt(deg)
    xw = x @ W
    out = xw * (dis * dis)[:, None]
    coef = dis[s] * dis[d] * w
    out = out.at[d].add(xw[s] * coef[:, None])
    return out + b


def _pad_i32(a, n, fill):
    return jnp.concatenate(
        [a.astype(jnp.int32), jnp.full((n - a.shape[0],), fill, jnp.int32)])


def _round_up(n, m):
    return ((n + m - 1) // m) * m


def kernel(x, edge_index, batch, W1, b1, p1, W2, b2, p2, Wb, bb,
           Wd1, bd1, Wd2, bd2, Wf, bf):
    N, _ = x.shape
    E = edge_index.shape[1]
    k1 = -(-N // 2)
    k2 = -(-k1 // 2)
    src = edge_index[0]
    dst = edge_index[1]

    NP0 = _round_up(N + 1, CHUNK)
    NP1 = _round_up(k1 + 1, CHUNK)
    NP2 = _round_up(k2 + 1, CHUNK)
    G0, G1, G2 = NP0 - 1, NP1 - 1, NP2 - 1
    EP = _round_up(E, NW * CHUNK)
    EPp1 = _round_up(k1, NW * CHUNK)
    EPp2 = _round_up(k2, NW * CHUNK)

    ones_n = {n: jnp.ones((n, 1), jnp.float32) for n in (N, k1)}

    d0 = _pad_i32(dst, EP, G0)
    s0 = _pad_i32(src, EP, 0)
    c0 = _edge_pass(16, NP0, EP, False)(d0)
    dis0 = lax.rsqrt(1.0 + (c0[0] + c0[1])[:N, :1])
    table1 = _mm_table((x,), None, W1, dis0)
    q1 = _edge_pass(64, NP0, EP, True)(table1, s0, d0)
    x1 = _combine(q1[0], q1[1], table1, dis0, b1, True)

    emask = jnp.ones(src.shape, bool)
    x1c = jax.nn.relu(_gcn_ctl(x, src, dst, emask, W1, b1))
    sc1 = (x1c @ p1) / jnp.linalg.norm(p1)
    vals1, perm1 = lax.top_k(sc1, k1)
    t1 = jnp.tanh(vals1)
    inv1 = jnp.full((N,), -1, src.dtype).at[perm1].set(
        jnp.arange(k1, dtype=src.dtype))
    s1r, d1r = inv1[src], inv1[dst]
    m1 = (s1r >= 0) & (d1r >= 0)
    s1 = jnp.where(m1, s1r, 0)
    d1 = jnp.where(m1, d1r, 0)
    es1 = _pad_i32(s1, EP, 0)
    ed1 = _pad_i32(jnp.where(m1, d1r, G1), EP, G1)

    pg1 = _edge_pass(64, NP1, EPp1, True)(
        x1, _pad_i32(perm1, EPp1, 0),
        _pad_i32(jnp.arange(k1, dtype=jnp.int32), EPp1, G1))
    c1 = _edge_pass(16, NP1, EP, False)(ed1)
    dis1 = lax.rsqrt(1.0 + (c1[0] + c1[1])[:k1, :1])
    table2 = _mm_table((pg1[0][:k1], pg1[1][:k1]), t1[:, None], W2, dis1)
    q2 = _edge_pass(128, NP1, EP, True)(table2, es1, ed1)
    x2 = _combine(q2[0], q2[1], table2, dis1, b2, True)

    x1pc = x1c[perm1] * jnp.tanh(vals1)[:, None]
    x2c = jax.nn.relu(_gcn_ctl(x1pc, s1, d1, m1, W2, b2))
    sc2 = (x2c @ p2) / jnp.linalg.norm(p2)
    vals2, perm2 = lax.top_k(sc2, k2)
    t2 = jnp.tanh(vals2)
    inv2 = jnp.full((k1,), -1, src.dtype).at[perm2].set(
        jnp.arange(k2, dtype=src.dtype))
    s2r, d2r = inv2[s1], inv2[d1]
    m2 = m1 & (s2r >= 0) & (d2r >= 0)
    es2 = _pad_i32(jnp.where(m2, s2r, 0), EP, 0)
    ed2_l3 = _pad_i32(jnp.where(m2, d2r, G2), EP, G2)
    ed2_d1 = _pad_i32(jnp.where(m2, d2r, G1), EP, G1)
    ed2_d2 = _pad_i32(jnp.where(m2, d2r, G0), EP, G0)

    pg2 = _edge_pass(128, NP2, EPp2, True)(
        x2, _pad_i32(perm2, EPp2, 0),
        _pad_i32(jnp.arange(k2, dtype=jnp.int32), EPp2, G2))
    c2 = _edge_pass(16, NP2, EP, False)(ed2_l3)
    cnt2 = (c2[0] + c2[1])[:k2, :1]
    dis3 = lax.rsqrt(1.0 + cnt2)
    disd1 = jnp.concatenate([dis3, jnp.ones((k1 - k2, 1))])
    disd2 = jnp.concatenate([dis3, jnp.ones((N - k2, 1))])

    table3 = _mm_table((pg2[0][:k2], pg2[1][:k2]), t2[:, None], Wb, dis3)
    q3 = _edge_pass(256, NP2, EP, True)(table3, es2, ed2_l3)
    x3 = _combine(q3[0], q3[1], table3, dis3, bb, True)

    u2 = _edge_pass(256, NP1, EPp2, True)(
        x3, _pad_i32(jnp.arange(k2, dtype=jnp.int32), EPp2, 0),
        _pad_i32(perm2, EPp2, G1))
    tabled1 = _mm_table((u2[0][:k1], u2[1][:k1]), None, Wd1, disd1)
    q4 = _edge_pass(128, NP1, EP, True)(tabled1, es2, ed2_d1)
    x3d = _combine(q4[0], q4[1], tabled1, disd1, bd1, True)

    u1 = _edge_pass(128, NP0, EPp1, True)(
        x3d, _pad_i32(jnp.arange(k1, dtype=jnp.int32), EPp1, 0),
        _pad_i32(perm1, EPp1, G0))
    tabled2 = _mm_table((u1[0][:N], u1[1][:N]), None, Wd2, disd2)
    q5 = _edge_pass(64, NP0, EP, True)(tabled2, es2, ed2_d2)
    x4d = _combine(q5[0], q5[1], tabled2, disd2, bd2, True)

    tablef = _mm_table((x4d,), None, Wf, disd2)
    q6 = _edge_pass(128, NP0, EP, True)(tablef, es2, ed2_d2)
    return _combine(q6[0], q6[1], tablef, disd2, bf, False)
